# baseline (speedup 1.0000x reference)
import functools

import jax
import jax.numpy as jnp
from jax import lax
from jax.experimental import pallas as pl
from jax.experimental.pallas import tpu as pltpu
from jax.experimental.pallas import tpu_sc as plsc

N, C, D = 10000, 32, 128
NW = 32
NS = 16
CH = 8
RPC = CH * C
NSTREAM = RPC // 128

CPW = 12
N_SC = NW * CPW * CH
NCHUNK = N_SC // CH
NQUAD = CPW // 4
LAST_BASE = NCHUNK - CPW
TC_B = 16
N_TC = N - N_SC
TC_NBLK = N_TC // TC_B


def _sc_body(in_hbm, out_hbm, buf0, buf1, idxs, zbuf,
             acc_sh, sin0, sin1, sadd, sout0, sout1, sout2, sout3,
             sz0, sz1, sz2, sz3):
    sid = lax.axis_index("s")
    wid = sid * 2 + lax.axis_index("c")
    base = jnp.minimum(wid * CPW, LAST_BASE)

    bufs = (buf0, buf1)
    sins = (sin0, sin1)
    souts = (sout0, sout1, sout2, sout3)
    szs = (sz0, sz1, sz2, sz3)

    acc_bases = tuple((sid * 4 + q) * CH for q in range(4))
    for q in range(4):
        for h in range(NSTREAM):
            for k in range(128 // 16):
                j = h * 128 + k * 16 + lax.iota(jnp.int32, 16)
                idxs[q, h, pl.ds(k * 16, 16)] = (
                    acc_bases[q] + lax.shift_right_logical(j, 5))

    zeros = jnp.zeros((16,), jnp.float32)
    for r in range(CH):
        for k in range(D // 16):
            zbuf[r, pl.ds(k * 16, 16)] = zeros

    def acc_rows(q):
        return acc_sh.at[pl.ds(acc_bases[q], CH)]

    def fire_in(k, t):
        pltpu.async_copy(in_hbm.at[pl.ds((base + k) * RPC, RPC)],
                         bufs[t % 2], sins[t % 2])

    def wait_in(t):
        pltpu.make_async_copy(
            in_hbm.at[pl.ds(0, RPC)], bufs[t % 2], sins[t % 2]).wait()

    def fire_out(k, t):
        pltpu.async_copy(acc_rows(t % 4),
                         out_hbm.at[pl.ds((base + k) * CH, CH)], souts[t % 4])

    def wait_out(t):
        pltpu.make_async_copy(
            acc_rows(t % 4), out_hbm.at[pl.ds(0, CH)], souts[t % 4]).wait()

    def fire_zero(t):
        pltpu.async_copy(zbuf, acc_rows(t % 4), szs[t % 4])

    def wait_zero(t):
        pltpu.make_async_copy(zbuf, acc_rows(t % 4), szs[t % 4]).wait()

    def fire_add(t):
        for h in range(NSTREAM):
            pltpu.async_copy(bufs[t % 2].at[pl.ds(h * 128, 128)],
                             acc_sh.at[idxs.at[t % 4, h]], sadd, add=True)

    def wait_add():
        for h in range(NSTREAM):
            pltpu.make_async_copy(bufs[0].at[pl.ds(h * 128, 128)],
                                  acc_sh.at[idxs.at[0, h]], sadd).wait()

    fire_in(0, 0)
    fire_in(1, 1)
    for q in range(4):
        fire_zero(q)

    def quad(m, carry):
        for t in range(4):
            k = 4 * m + t

            @pl.when(k >= 1)
            def _():
                wait_add()

            @pl.when(jnp.logical_and(k >= 1, k <= CPW - 2))
            def _():
                fire_in(k + 1, t + 1)

            wait_in(t)

            @pl.when(k >= 3)
            def _():
                wait_out(t + 1)

            @pl.when(jnp.logical_and(k >= 3, k <= CPW - 2))
            def _():
                fire_zero(t + 1)

            wait_zero(t)

            fire_add(t)

            @pl.when(k >= 2)
            def _():
                fire_out(k - 2, t + 2)

        return carry

    lax.fori_loop(0, NQUAD, quad, 0)

    wait_add()
    pl.delay(2000)
    fire_out(CPW - 2, CPW - 2)
    fire_out(CPW - 1, CPW - 1)
    wait_out(CPW - 3)
    wait_out(CPW - 2)
    wait_out(CPW - 1)


def _tc_body(x_ref, o_ref):
    o_ref[...] = jnp.sum(x_ref[...], axis=1)


@jax.jit
def _sum_children(neighbour_states):
    flat = neighbour_states.reshape(N * C, D)
    mesh = plsc.VectorSubcoreMesh(core_axis_name="c", subcore_axis_name="s")
    kern = functools.partial(
        pl.kernel,
        out_type=jax.ShapeDtypeStruct((N_SC, D), jnp.float32),
        mesh=mesh,
        scratch_types=[
            pltpu.VMEM((RPC, D), jnp.float32),
            pltpu.VMEM((RPC, D), jnp.float32),
            pltpu.VMEM((4, NSTREAM, 128), jnp.int32),
            pltpu.VMEM((CH, D), jnp.float32),
            pltpu.VMEM_SHARED((NS * 4 * CH, D), jnp.float32),
            pltpu.SemaphoreType.DMA,
            pltpu.SemaphoreType.DMA,
            pltpu.SemaphoreType.DMA,
            pltpu.SemaphoreType.DMA,
            pltpu.SemaphoreType.DMA,
            pltpu.SemaphoreType.DMA,
            pltpu.SemaphoreType.DMA,
            pltpu.SemaphoreType.DMA,
            pltpu.SemaphoreType.DMA,
            pltpu.SemaphoreType.DMA,
            pltpu.SemaphoreType.DMA,
        ],
    )(_sc_body)
    sc_out = kern(flat)

    tc_out = pl.pallas_call(
        _tc_body,
        grid=(TC_NBLK,),
        in_specs=[pl.BlockSpec((TC_B, C, D), lambda i: (N_SC // TC_B + i, 0, 0))],
        out_specs=pl.BlockSpec((TC_B, D), lambda i: (N_SC // TC_B + i, 0)),
        out_shape=jax.ShapeDtypeStruct((N, D), jnp.float32),
    )(neighbour_states)

    return lax.dynamic_update_slice(tc_out, sc_out, (0, 0))


def kernel(neighbour_states):
    return _sum_children(neighbour_states)


# speedup vs baseline: 2.6847x; 2.6847x over previous
import jax
import jax.numpy as jnp
from jax.experimental import pallas as pl

N, C, D = 10000, 32, 128
B = 80
NBLK = N // B


def _tc_body(x_ref, o_ref):
    o_ref[...] = jnp.sum(x_ref[...], axis=1)


@jax.jit
def _sum_children(neighbour_states):
    return pl.pallas_call(
        _tc_body,
        grid=(NBLK,),
        in_specs=[pl.BlockSpec((B, C, D), lambda i: (i, 0, 0))],
        out_specs=pl.BlockSpec((B, D), lambda i: (i, 0)),
        out_shape=jax.ShapeDtypeStruct((N, D), jnp.float32),
    )(neighbour_states)


def kernel(neighbour_states):
    return _sum_children(neighbour_states)


# speedup vs baseline: 5.0249x; 1.8717x over previous
import jax
import jax.numpy as jnp
from jax.experimental import pallas as pl

N, C, D = 10000, 32, 128
B = 400
NBLK = N // B


def _tc_body(x_ref, o_ref):
    x = x_ref[...]
    s = C // 2
    while s >= 1:
        x = x[:, :s, :] + x[:, s:, :]
        s //= 2
    o_ref[...] = x[:, 0, :]


@jax.jit
def _sum_children(neighbour_states):
    return pl.pallas_call(
        _tc_body,
        grid=(NBLK,),
        in_specs=[pl.BlockSpec((B, C, D), lambda i: (i, 0, 0))],
        out_specs=pl.BlockSpec((B, D), lambda i: (i, 0)),
        out_shape=jax.ShapeDtypeStruct((N, D), jnp.float32),
    )(neighbour_states)


def kernel(neighbour_states):
    return _sum_children(neighbour_states)
